# baseline (speedup 1.0000x reference)
import numpy as np
import jax
import jax.numpy as jnp
from jax.experimental import pallas as pl
from jax.experimental.pallas import tpu as pltpu

B = 4096
SEQ = 25
SEQP = 32
D = 128
NH = 2
HD = 64
NL = 3
IN_DIM = 8
ENC = 64
NEG = -1e9
FAR = -2e9
BB = 8
M = BB * SEQP


def _adj25():
    adj = np.zeros((SEQ, SEQ), dtype=bool)
    for r in range(5):
        for c in range(5):
            x = 5 * r + c
            for dr, dc in ((-1, 0), (1, 0), (0, -1), (0, 1)):
                nr, nc = r + dr, c + dc
                if 0 <= nr < 5 and 0 <= nc < 5:
                    adj[x, 5 * nr + nc] = True
    return adj


_ADJ = jnp.asarray(_adj25())


def _ln(x, s, b):
    m = jnp.mean(x, axis=-1, keepdims=True)
    d = x - m
    v = jnp.mean(d * d, axis=-1, keepdims=True)
    return d * jax.lax.rsqrt(v + 1e-5) * s + b


def _bf(x):
    return x.astype(jnp.bfloat16)


def _body(xp_ref, env_ref, mask_ref, gw_ref, gb_ref, wqkv_ref, wo_ref,
          l1s_ref, l1b_ref, ffw_ref, ffb_ref, l2s_ref, l2b_ref,
          ns_ref, nb_ref, ew1_ref, eb1_ref, ew2_ref, eb2_ref,
          dw1a_ref, dw1b_ref, db1_ref, dw2_ref, db2_ref, out_ref):
    f32 = jnp.float32
    x = xp_ref[...]
    hb = jax.nn.relu(
        jnp.dot(_bf(x), _bf(gw_ref[...]), preferred_element_type=f32)
        + gb_ref[...])
    mask = mask_ref[...]
    for l in range(NL):
        hbb = _bf(hb)
        qkv = jnp.dot(hbb, _bf(wqkv_ref[l]), preferred_element_type=f32)
        o_heads = []
        for h in range(NH):
            q = qkv[:, h * HD:(h + 1) * HD]
            k = qkv[:, D + h * HD:D + (h + 1) * HD]
            v = qkv[:, 2 * D + h * HD:2 * D + (h + 1) * HD]
            s = jax.lax.dot_general(
                _bf(q), _bf(k), (((1,), (1,)), ((), ())),
                preferred_element_type=f32)
            s = s * 0.125 + mask
            mx = jnp.max(s, axis=-1, keepdims=True)
            p = jnp.exp(s - mx)
            a = p / jnp.sum(p, axis=-1, keepdims=True)
            o_heads.append(
                jnp.dot(_bf(a), _bf(v), preferred_element_type=f32))
        o = jnp.concatenate(o_heads, axis=-1)
        proj = jnp.dot(_bf(o), _bf(wo_ref[l]), preferred_element_type=f32)
        hb = _ln(hb + proj, l1s_ref[l], l1b_ref[l])
        ff = jax.nn.relu(
            jnp.dot(_bf(hb), _bf(ffw_ref[l]), preferred_element_type=f32)
            + ffb_ref[l])
        hb = _ln(hb + ff, l2s_ref[l], l2b_ref[l])
    rep = _ln(hb, ns_ref[...], nb_ref[...])
    out1 = jnp.dot(_bf(rep), _bf(dw1a_ref[...]), preferred_element_type=f32)
    e = env_ref[...]
    h1 = jax.nn.relu(e[:, 0:1] * ew1_ref[0:1, :]
                     + e[:, 1:2] * ew1_ref[1:2, :] + eb1_ref[...])
    h2 = jax.nn.relu(
        jnp.dot(_bf(h1), _bf(ew2_ref[...]), preferred_element_type=f32)
        + eb2_ref[...])
    out2 = jnp.dot(_bf(h2), _bf(dw1b_ref[...]), preferred_element_type=f32)
    y = jax.nn.relu(out1.reshape(BB, SEQP, 64)
                    + out2[:, None, :] + db1_ref[...])
    z = jnp.sum(y * dw2_ref[...], axis=-1) + db2_ref[...]
    out_ref[...] = z[:, :SEQ]


def kernel(structure_body, structure_connect, obs_env, local_obs, params):
    del structure_connect
    p = params
    f32 = jnp.float32
    body_f = structure_body.reshape(-1)
    R = jnp.where(_ADJ & (body_f != 0)[None, :], 0.0, NEG).astype(f32)
    R32 = jnp.full((SEQP, SEQP), NEG, f32)
    R32 = R32.at[:SEQ, :SEQ].set(R)
    R32 = R32.at[:SEQ, SEQ:].set(FAR)
    bi = jnp.arange(256) // SEQP
    mask256 = jnp.where(bi[:, None] == bi[None, :],
                        jnp.tile(R32, (BB, BB)), FAR).astype(f32)

    xp = jnp.pad(local_obs, ((0, 0), (0, SEQP - SEQ), (0, 0))
                 ).reshape(B * SEQP, IN_DIM)
    wqkv = jnp.concatenate([p['wq'], p['wk'], p['wv']], axis=-1)
    gb = p['gnn_in_b'].reshape(1, D)
    l1s = p['ln1_s'].reshape(NL, 1, D)
    l1b = p['ln1_b'].reshape(NL, 1, D)
    ffb = p['ffb'].reshape(NL, 1, D)
    l2s = p['ln2_s'].reshape(NL, 1, D)
    l2b = p['ln2_b'].reshape(NL, 1, D)
    ns = p['norm_s'].reshape(1, D)
    nb = p['norm_b'].reshape(1, D)
    eb1 = p['enc_b1'].reshape(1, ENC)
    eb2 = p['enc_b2'].reshape(1, ENC)
    dw1a = p['dec_w1'][:D]
    dw1b = p['dec_w1'][D:]
    db1 = p['dec_b1'].reshape(1, 64)
    dw2 = p['dec_w2'].reshape(1, 64)
    db2 = p['dec_b2'].reshape(1, 1)

    grid = (B // BB,)
    zero2 = lambda i: (0, 0)
    zero3 = lambda i: (0, 0, 0)
    in_specs = [
        pl.BlockSpec((M, IN_DIM), lambda i: (i, 0)),
        pl.BlockSpec((BB, 2), lambda i: (i, 0)),
        pl.BlockSpec((256, 256), zero2),
        pl.BlockSpec((IN_DIM, D), zero2),
        pl.BlockSpec((1, D), zero2),
        pl.BlockSpec((NL, D, 3 * D), zero3),
        pl.BlockSpec((NL, D, D), zero3),
        pl.BlockSpec((NL, 1, D), zero3),
        pl.BlockSpec((NL, 1, D), zero3),
        pl.BlockSpec((NL, D, D), zero3),
        pl.BlockSpec((NL, 1, D), zero3),
        pl.BlockSpec((NL, 1, D), zero3),
        pl.BlockSpec((NL, 1, D), zero3),
        pl.BlockSpec((1, D), zero2),
        pl.BlockSpec((1, D), zero2),
        pl.BlockSpec((2, ENC), zero2),
        pl.BlockSpec((1, ENC), zero2),
        pl.BlockSpec((ENC, ENC), zero2),
        pl.BlockSpec((1, ENC), zero2),
        pl.BlockSpec((D, 64), zero2),
        pl.BlockSpec((ENC, 64), zero2),
        pl.BlockSpec((1, 64), zero2),
        pl.BlockSpec((1, 64), zero2),
        pl.BlockSpec((1, 1), zero2),
    ]
    out2d = pl.pallas_call(
        _body,
        grid=grid,
        in_specs=in_specs,
        out_specs=pl.BlockSpec((BB, SEQ), lambda i: (i, 0)),
        out_shape=jax.ShapeDtypeStruct((B, SEQ), f32),
        compiler_params=pltpu.CompilerParams(
            dimension_semantics=("parallel",)),
    )(xp, obs_env, mask256, p['gnn_in_w'], gb, wqkv, p['wo'],
      l1s, l1b, p['ffw'], ffb, l2s, l2b, ns, nb,
      p['enc_w1'], eb1, p['enc_w2'], eb2, dw1a, dw1b, db1, dw2, db2)
    return jnp.transpose(out2d)[:, :, None]
